# baseline (speedup 1.0000x reference)
import jax
import jax.numpy as jnp
from jax.experimental import pallas as pl

N = 10000
R = 8


def _identity_body(x_ref, o_ref):
    o_ref[...] = x_ref[...]


def _layer(X, src, dst, et, W, Wself):
    seg = dst * R + et
    msgs = jnp.take(X, src, axis=0)
    agg = jax.ops.segment_sum(msgs, seg, num_segments=N * R)
    cnt = jax.ops.segment_sum(jnp.ones((src.shape[0],), X.dtype), seg, num_segments=N * R)
    agg = agg / jnp.clip(cnt, 1.0)[:, None]
    agg = agg.reshape(N, R, X.shape[1])
    return jnp.einsum('nrd,rde->ne', agg, W) + X @ Wself


def kernel(X, edge_index, edge_type, W1, Wself1, W2, Wself2, epoch):
    src, dst = edge_index[0], edge_index[1]
    h = jax.nn.relu(_layer(X, src, dst, edge_type, W1, Wself1))
    out = _layer(h, src, dst, edge_type, W2, Wself2)
    out = pl.pallas_call(
        _identity_body,
        out_shape=jax.ShapeDtypeStruct(out.shape, out.dtype),
    )(out)
    return out


# speedup vs baseline: 2.9250x; 2.9250x over previous
import functools

import jax
import jax.numpy as jnp
from jax import lax
from jax.experimental import pallas as pl
from jax.experimental.pallas import tpu as pltpu
from jax.experimental.pallas import tpu_sc as plsc

N = 10000
E = 320000
R = 8
D = 128

NC = 2
NS = 16
LANES = 128
ROWS_PER_TILE = 80
EP = NC * NS * ROWS_PER_TILE * LANES
EROWS = EP // LANES

NPAD = 10240
ACC_TILE_ROWS = NPAD // NS
CNT_BINS = N * R + 8
CNT_PAD = ((CNT_BINS + NS * 128 - 1) // (NS * 128)) * (NS * 128)
CNT_TILE = CNT_PAD // NS

_MESH = plsc.VectorSubcoreMesh(core_axis_name="c", subcore_axis_name="s")


def _sc_wid():
    c = lax.axis_index("c")
    s = lax.axis_index("s")
    return c, s, c * NS + s


def _sc_cnt_w_body(seg_hbm, zeros_hbm, w_hbm, cnt_sh, seg_v, ones_v, vals_v, w_v):
    c, s, wid = _sc_wid()

    def fill_ones(g, _):
        ones_v[pl.ds(g * 16, 16)] = jnp.ones((16,), jnp.float32)
        return 0
    lax.fori_loop(0, LANES // 16, fill_ones, 0)

    pltpu.sync_copy(zeros_hbm, cnt_sh.at[pl.ds(s * CNT_TILE, CNT_TILE)])
    plsc.subcore_barrier()

    rows_per_tile_a = EROWS // NS
    pltpu.sync_copy(seg_hbm.at[pl.ds(s * rows_per_tile_a, rows_per_tile_a)],
                    seg_v.at[pl.ds(0, rows_per_tile_a)])

    def hist_step(j, _):
        pltpu.sync_copy(ones_v, cnt_sh.at[seg_v.at[j]], add=True)
        return 0
    lax.fori_loop(0, rows_per_tile_a, hist_step, 0)
    plsc.subcore_barrier()

    base = wid * ROWS_PER_TILE
    pltpu.sync_copy(seg_hbm.at[pl.ds(base, ROWS_PER_TILE)],
                    seg_v.at[pl.ds(0, ROWS_PER_TILE)])

    def w_step(j, _):
        pltpu.sync_copy(cnt_sh.at[seg_v.at[j]], vals_v)
        def grp(g, _):
            cv = vals_v[pl.ds(g * 16, 16)]
            w_v[j, pl.ds(g * 16, 16)] = 1.0 / jnp.maximum(cv, 1.0)
            return 0
        lax.fori_loop(0, LANES // 16, grp, 0)
        return 0
    lax.fori_loop(0, ROWS_PER_TILE, w_step, 0)
    pltpu.sync_copy(w_v, w_hbm.at[pl.ds(base, ROWS_PER_TILE)])


def _sc_cnt_w(seg2d, zeros_cnt):
    return pl.kernel(
        _sc_cnt_w_body,
        out_type=jax.ShapeDtypeStruct((EROWS, LANES), jnp.float32),
        mesh=_MESH,
        scratch_types=[
            pltpu.VMEM_SHARED((CNT_PAD,), jnp.float32),
            pltpu.VMEM((EROWS // NS, LANES), jnp.int32),
            pltpu.VMEM((LANES,), jnp.float32),
            pltpu.VMEM((LANES,), jnp.float32),
            pltpu.VMEM((ROWS_PER_TILE, LANES), jnp.float32),
        ],
        name="sc_hist_w",
    )(seg2d, zeros_cnt)


def _sc_agg_body(y_hbm, gidx_hbm, dst_hbm, w_hbm, zeros_hbm, p_hbm,
                 acc_sh, gidx_v, dst_v, w_v, row_v):
    c, s, wid = _sc_wid()

    pltpu.sync_copy(zeros_hbm, acc_sh.at[pl.ds(s * ACC_TILE_ROWS, ACC_TILE_ROWS)])
    plsc.subcore_barrier()

    base = wid * ROWS_PER_TILE
    pltpu.sync_copy(gidx_hbm.at[pl.ds(base, ROWS_PER_TILE)], gidx_v)
    pltpu.sync_copy(dst_hbm.at[pl.ds(base, ROWS_PER_TILE)], dst_v)
    pltpu.sync_copy(w_hbm.at[pl.ds(base, ROWS_PER_TILE)], w_v)

    def edge_step(j, _):
        pltpu.sync_copy(y_hbm.at[gidx_v.at[j]], row_v)

        def grp(g, _):
            w16 = w_v[j, pl.ds(g * 16, 16)]
            for l in range(16):
                ws = w16[l]
                e = g * 16 + l
                for v in range(D // 16):
                    row_v[e, pl.ds(v * 16, 16)] = row_v[e, pl.ds(v * 16, 16)] * ws
            return 0
        lax.fori_loop(0, LANES // 16, grp, 0)

        pltpu.sync_copy(row_v, acc_sh.at[dst_v.at[j]], add=True)
        return 0
    lax.fori_loop(0, ROWS_PER_TILE, edge_step, 0)
    plsc.subcore_barrier()

    out_base = c * NPAD + s * ACC_TILE_ROWS
    pltpu.sync_copy(acc_sh.at[pl.ds(s * ACC_TILE_ROWS, ACC_TILE_ROWS)],
                    p_hbm.at[pl.ds(out_base, ACC_TILE_ROWS)])


def _sc_agg(y2d, gidx2d, dst2d, w2d, zeros_acc):
    return pl.kernel(
        _sc_agg_body,
        out_type=jax.ShapeDtypeStruct((NC * NPAD, D), jnp.float32),
        mesh=_MESH,
        scratch_types=[
            pltpu.VMEM_SHARED((NPAD, D), jnp.float32),
            pltpu.VMEM((ROWS_PER_TILE, LANES), jnp.int32),
            pltpu.VMEM((ROWS_PER_TILE, LANES), jnp.int32),
            pltpu.VMEM((ROWS_PER_TILE, LANES), jnp.float32),
            pltpu.VMEM((LANES, D), jnp.float32),
        ],
        name="sc_gather_scale_scatter",
    )(y2d, gidx2d, dst2d, w2d, zeros_acc)


BN = 1000


def _tc_y_body(x_ref, w_ref, y_ref):
    x = x_ref[...]
    for r in range(R):
        y_ref[r] = jnp.dot(x, w_ref[r], preferred_element_type=jnp.float32)


def _tc_y(X, W):
    return pl.pallas_call(
        _tc_y_body,
        grid=(N // BN,),
        in_specs=[
            pl.BlockSpec((BN, D), lambda i: (i, 0)),
            pl.BlockSpec((R, D, D), lambda i: (0, 0, 0)),
        ],
        out_specs=pl.BlockSpec((R, BN, D), lambda i: (0, i, 0)),
        out_shape=jax.ShapeDtypeStruct((R, N, D), jnp.float32),
    )(X, W)


def _tc_mid_body(x_ref, ws_ref, p0_ref, p1_ref, w2_ref, h_ref, y_ref):
    x = x_ref[...]
    h = p0_ref[...] + p1_ref[...] + jnp.dot(x, ws_ref[...],
                                            preferred_element_type=jnp.float32)
    h = jnp.maximum(h, 0.0)
    h_ref[...] = h
    for r in range(R):
        y_ref[r] = jnp.dot(h, w2_ref[r], preferred_element_type=jnp.float32)


def _tc_mid(X, Wself1, P0, P1, W2):
    return pl.pallas_call(
        _tc_mid_body,
        grid=(N // BN,),
        in_specs=[
            pl.BlockSpec((BN, D), lambda i: (i, 0)),
            pl.BlockSpec((D, D), lambda i: (0, 0)),
            pl.BlockSpec((BN, D), lambda i: (i, 0)),
            pl.BlockSpec((BN, D), lambda i: (i, 0)),
            pl.BlockSpec((R, D, D), lambda i: (0, 0, 0)),
        ],
        out_specs=[
            pl.BlockSpec((BN, D), lambda i: (i, 0)),
            pl.BlockSpec((R, BN, D), lambda i: (0, i, 0)),
        ],
        out_shape=[
            jax.ShapeDtypeStruct((N, D), jnp.float32),
            jax.ShapeDtypeStruct((R, N, D), jnp.float32),
        ],
    )(X, Wself1, P0, P1, W2)


def _tc_final_body(h_ref, ws_ref, p0_ref, p1_ref, o_ref):
    o_ref[...] = p0_ref[...] + p1_ref[...] + jnp.dot(
        h_ref[...], ws_ref[...], preferred_element_type=jnp.float32)


def _tc_final(h, Wself2, P0, P1):
    return pl.pallas_call(
        _tc_final_body,
        grid=(N // BN,),
        in_specs=[
            pl.BlockSpec((BN, D), lambda i: (i, 0)),
            pl.BlockSpec((D, D), lambda i: (0, 0)),
            pl.BlockSpec((BN, D), lambda i: (i, 0)),
            pl.BlockSpec((BN, D), lambda i: (i, 0)),
        ],
        out_specs=pl.BlockSpec((BN, D), lambda i: (i, 0)),
        out_shape=jax.ShapeDtypeStruct((N, D), jnp.float32),
    )(h, Wself2, P0, P1)


def kernel(X, edge_index, edge_type, W1, Wself1, W2, Wself2, epoch):
    src = edge_index[0]
    dst = edge_index[1]
    et = edge_type

    pad = EP - E
    gidx = jnp.pad(et * N + src, (0, pad)).reshape(EROWS, LANES)
    dstp = jnp.pad(dst, (0, pad), constant_values=N).reshape(EROWS, LANES)
    seg = jnp.pad(dst * R + et, (0, pad), constant_values=N * R).reshape(EROWS, LANES)

    zeros_cnt = jnp.zeros((CNT_TILE,), jnp.float32)
    zeros_acc = jnp.zeros((ACC_TILE_ROWS, D), jnp.float32)

    w2d = _sc_cnt_w(seg, zeros_cnt)

    Y1 = _tc_y(X, W1).reshape(R * N, D)
    P = _sc_agg(Y1, gidx, dstp, w2d, zeros_acc)
    P0 = lax.slice(P, (0, 0), (N, D))
    P1 = lax.slice(P, (NPAD, 0), (NPAD + N, D))

    h, Y2 = _tc_mid(X, Wself1, P0, P1, W2)
    Y2 = Y2.reshape(R * N, D)
    Q = _sc_agg(Y2, gidx, dstp, w2d, zeros_acc)
    Q0 = lax.slice(Q, (0, 0), (N, D))
    Q1 = lax.slice(Q, (NPAD, 0), (NPAD + N, D))

    return _tc_final(h, Wself2, Q0, Q1)
